# baseline (speedup 1.0000x reference)
import functools

import jax
import jax.numpy as jnp
from jax.experimental import pallas as pl
from jax.experimental.pallas import tpu as pltpu

_STEPS = 4
_MXTOT = 2560


def _split2_bf16(x):
    h1 = x.astype(jnp.bfloat16)
    r1 = x - h1.astype(jnp.float32)
    h2 = r1.astype(jnp.bfloat16)
    return h1, h2


def _prepare(gts, preds):
    b, n, _ = gts.shape
    m = preds.shape[1]
    bf16 = jnp.bfloat16
    f32 = jnp.float32

    gb = gts.astype(bf16)
    pb = preds.astype(bf16)
    g2 = jnp.sum(gts * gts, axis=-1)
    p2 = jnp.sum(preds * preds, axis=-1)
    g2a, g2b = _split2_bf16(g2)
    p2a, p2b = _split2_bf16(p2)

    ones_n = jnp.ones((b, n), bf16)
    ones_m = jnp.ones((b, m), bf16)

    g_aug = jnp.stack(
        [-2.0 * gb[..., 0], -2.0 * gb[..., 1], -2.0 * gb[..., 2],
         g2a, g2b,
         ones_n, ones_n, ones_n],
        axis=-1)
    p_aug = jnp.stack(
        [pb[..., 0], pb[..., 1], pb[..., 2],
         ones_m, ones_m,
         p2a, p2b, jnp.zeros((b, m), bf16)],
        axis=1)

    gr = jnp.concatenate(
        [-2.0 * gb.astype(f32), g2[..., None]], axis=-1)
    pr = jnp.stack(
        [pb[..., 0].astype(f32), pb[..., 1].astype(f32),
         pb[..., 2].astype(f32), p2], axis=1)

    p_aug_m = p_aug[:, :, :_MXTOT]
    pr_v = pr[:, :, _MXTOT:]
    return g_aug, p_aug_m, gr, pr_v


def _chamfer_blk(g_ref, p_ref, gr_ref, pr_ref, out_ref, minx_ref, sumy_ref,
                 *, n_steps, m_total):
    s = pl.program_id(1)

    g = g_ref[0]
    p = p_ref[0]

    d1 = jnp.dot(g, p, preferred_element_type=jnp.float32)
    minx1 = jnp.min(d1, axis=1, keepdims=True)
    miny1 = jnp.min(d1, axis=0, keepdims=True)

    gr = gr_ref[0]
    pr = pr_ref[0]
    gx2 = gr[:, 0:1]
    gy2 = gr[:, 1:2]
    gz2 = gr[:, 2:3]
    g2 = gr[:, 3:4]
    pxb = pr[0:1, :]
    pyb = pr[1:2, :]
    pzb = pr[2:3, :]
    p2 = pr[3:4, :]
    d2 = g2 + (p2 + (gx2 * pxb + gy2 * pyb + gz2 * pzb))
    minx2 = jnp.min(d2, axis=1, keepdims=True)
    miny2 = jnp.min(d2, axis=0, keepdims=True)

    blk_minx = jnp.minimum(minx1, minx2)
    sy = (jnp.sum(jnp.maximum(miny1, 0.0))
          + jnp.sum(jnp.maximum(miny2, 0.0)))

    @pl.when(s == 0)
    def _init():
        minx_ref[...] = blk_minx
        sumy_ref[0, 0] = sy

    @pl.when(s > 0)
    def _acc():
        minx_ref[...] = jnp.minimum(minx_ref[...], blk_minx)
        sumy_ref[0, 0] = sumy_ref[0, 0] + sy

    @pl.when(s == n_steps - 1)
    def _fin():
        n = g.shape[0]
        sum_x = jnp.sum(jnp.maximum(minx_ref[...], 0.0))
        val = sum_x / n + sumy_ref[0, 0] / m_total
        out_ref[...] = jnp.full((1, 1, 128), val, jnp.float32)


def kernel(gts, preds):
    b, n, _ = gts.shape
    m = preds.shape[1]
    g_aug, p_aug_m, gr, pr_v = _prepare(gts, preds)
    mx = _MXTOT // _STEPS
    mv = (m - _MXTOT) // _STEPS

    out = pl.pallas_call(
        functools.partial(_chamfer_blk, n_steps=_STEPS, m_total=m),
        grid=(b, _STEPS),
        in_specs=[
            pl.BlockSpec((1, n, 8), lambda i, j: (i, 0, 0)),
            pl.BlockSpec((1, 8, mx), lambda i, j: (i, 0, j)),
            pl.BlockSpec((1, n, 4), lambda i, j: (i, 0, 0)),
            pl.BlockSpec((1, 4, mv), lambda i, j: (i, 0, j)),
        ],
        out_specs=pl.BlockSpec((1, 1, 128), lambda i, j: (i, 0, 0)),
        out_shape=jax.ShapeDtypeStruct((b, 1, 128), jnp.float32),
        scratch_shapes=[
            pltpu.VMEM((n, 1), jnp.float32),
            pltpu.SMEM((1, 1), jnp.float32),
        ],
    )(g_aug, p_aug_m, gr, pr_v)
    return jnp.mean(out[:, 0, 0])
